# baseline (speedup 1.0000x reference)
import functools

import jax
import jax.numpy as jnp
from jax import lax
from jax.experimental import pallas as pl
from jax.experimental.pallas import tpu as pltpu
from jax.experimental.pallas import tpu_sc as plsc

NC = 2
NS = 16
NW = NC * NS
LN = 16
CH = 128
XW = 16
XG = 128

_f32 = jnp.float32
_i32 = jnp.int32


def _silu(v):
    return v * jax.nn.sigmoid(v)


def _dbf(a, b):
    return jnp.dot(a.astype(jnp.bfloat16), b.astype(jnp.bfloat16),
                   preferred_element_type=_f32)


def _tr(v):
    return v.astype(jnp.bfloat16).astype(_f32)


def _full_spec(shape):
    nd = len(shape)
    return pl.BlockSpec(shape, lambda i, _nd=nd: (0,) * _nd)


def _embed(h, W_in, b_in, A0, B0, be10, BN=1000):
    N, IN = h.shape
    H = W_in.shape[1]

    def body(h_ref, wi_ref, bi_ref, a_ref, b_ref, be_ref, he_ref, hs_ref, hd_ref):
        he = _dbf(h_ref[...], wi_ref[...]) + bi_ref[...]
        he_ref[...] = he
        hs_ref[...] = _dbf(he, a_ref[...]) + be_ref[...]
        hd_ref[...] = _dbf(he, b_ref[...])

    return pl.pallas_call(
        body,
        grid=(N // BN,),
        in_specs=[
            pl.BlockSpec((BN, IN), lambda i: (i, 0)),
            _full_spec((IN, H)),
            _full_spec((1, H)),
            _full_spec((H, H)),
            _full_spec((H, H)),
            _full_spec((1, H)),
        ],
        out_specs=[
            pl.BlockSpec((BN, H), lambda i: (i, 0)),
            pl.BlockSpec((BN, H), lambda i: (i, 0)),
            pl.BlockSpec((BN, H), lambda i: (i, 0)),
        ],
        out_shape=[jax.ShapeDtypeStruct((N, H), _f32)] * 3,
        compiler_params=pltpu.CompilerParams(dimension_semantics=("parallel",)),
    )(h, W_in, b_in, A0, B0, be10)


def _edge_mlp(hsg, hdg, xr, We2, be2, wc, wx, bx, BE=1000):
    E, H = hsg.shape

    def body(hs_ref, hd_ref, xr_ref, w2_ref, b2_ref, wc_ref, wx_ref, bx_ref,
             m_ref, rc_ref):
        xrv = xr_ref[...]
        d2 = xrv[:, 3:4]
        pre = hs_ref[...] + hd_ref[...] + _tr(d2) * _tr(wc_ref[...])
        m1 = _silu(pre)
        m2 = _silu(_dbf(m1, w2_ref[...]) + b2_ref[...])
        m_ref[...] = m2
        t = jnp.sum(_tr(m2) * _tr(wx_ref[...]), axis=1, keepdims=True) + bx_ref[...]
        coef = jnp.tanh(t)
        BE_ = xrv.shape[0]
        rc_ref[...] = jnp.concatenate(
            [xrv[:, :3] * coef, jnp.ones((BE_, 1), _f32),
             jnp.zeros((BE_, XG - 4), _f32)], axis=1)

    return pl.pallas_call(
        body,
        grid=(E // BE,),
        in_specs=[
            pl.BlockSpec((BE, H), lambda i: (i, 0)),
            pl.BlockSpec((BE, H), lambda i: (i, 0)),
            pl.BlockSpec((BE, 4), lambda i: (i, 0)),
            _full_spec((H, H)),
            _full_spec((1, H)),
            _full_spec((1, H)),
            _full_spec((1, H)),
            _full_spec((1, 1)),
        ],
        out_specs=[
            pl.BlockSpec((BE, H), lambda i: (i, 0)),
            pl.BlockSpec((BE, XG), lambda i: (i, 0)),
        ],
        out_shape=[
            jax.ShapeDtypeStruct((E, H), _f32),
            jax.ShapeDtypeStruct((E, XG), _f32),
        ],
        compiler_params=pltpu.CompilerParams(dimension_semantics=("parallel",)),
    )(hsg, hdg, xr, We2, be2, wc, wx, bx)


def _node_update(hres, agg, xp, xaggp, Wn1a, Wn1b, bn1, Wn2, bn2, Wnext, last,
                 BN=1000):
    N, H = hres.shape

    def _common(h_ref, agg_ref, x_ref, xa_ref, wa_ref, wb_ref, b1_ref, w2_ref,
                b2_ref):
        hres_v = h_ref[...]
        hn1 = _silu(_dbf(hres_v, wa_ref[...])
                    + _dbf(agg_ref[...], wb_ref[...])
                    + b1_ref[...])
        hn2 = _dbf(hn1, w2_ref[...]) + b2_ref[...]
        hnew = _silu(hn2) + hres_v
        xa = xa_ref[0] + xa_ref[1]
        inv = 1.0 / (xa[:, 3:4] + 1.0)
        col = lax.broadcasted_iota(_i32, (1, XG), 1)
        mask3 = (col < 3).astype(_f32)
        xnew = x_ref[...] + xa * inv * mask3
        return hnew, xnew

    if last:
        W_out, b_out = Wnext
        OUT = W_out.shape[1]

        def body(h_ref, agg_ref, x_ref, xa_ref, wa_ref, wb_ref, b1_ref, w2_ref,
                 b2_ref, wo_ref, bo_ref, ho_ref, xo_ref):
            hnew, xnew = _common(h_ref, agg_ref, x_ref, xa_ref, wa_ref, wb_ref,
                                 b1_ref, w2_ref, b2_ref)
            xo_ref[...] = xnew
            ho_ref[...] = _dbf(hnew, wo_ref[...]) + bo_ref[...]

        extra_in = [_full_spec((H, OUT)), _full_spec((1, OUT))]
        extra_args = [W_out, b_out]
        out_specs = [
            pl.BlockSpec((BN, OUT), lambda i: (i, 0)),
            pl.BlockSpec((BN, XG), lambda i: (i, 0)),
        ]
        out_shape = [
            jax.ShapeDtypeStruct((N, OUT), _f32),
            jax.ShapeDtypeStruct((N, XG), _f32),
        ]
    else:
        An, Bn, ben = Wnext

        def body(h_ref, agg_ref, x_ref, xa_ref, wa_ref, wb_ref, b1_ref, w2_ref,
                 b2_ref, an_ref, bnn_ref, ben_ref, hn_ref, xo_ref, hs_ref, hd_ref):
            hnew, xnew = _common(h_ref, agg_ref, x_ref, xa_ref, wa_ref, wb_ref,
                                 b1_ref, w2_ref, b2_ref)
            hn_ref[...] = hnew
            xo_ref[...] = xnew
            hs_ref[...] = _dbf(hnew, an_ref[...]) + ben_ref[...]
            hd_ref[...] = _dbf(hnew, bnn_ref[...])

        extra_in = [_full_spec((H, H)), _full_spec((H, H)), _full_spec((1, H))]
        extra_args = [An, Bn, ben]
        out_specs = [
            pl.BlockSpec((BN, H), lambda i: (i, 0)),
            pl.BlockSpec((BN, XG), lambda i: (i, 0)),
            pl.BlockSpec((BN, H), lambda i: (i, 0)),
            pl.BlockSpec((BN, H), lambda i: (i, 0)),
        ]
        out_shape = [
            jax.ShapeDtypeStruct((N, H), _f32),
            jax.ShapeDtypeStruct((N, XG), _f32),
            jax.ShapeDtypeStruct((N, H), _f32),
            jax.ShapeDtypeStruct((N, H), _f32),
        ]

    return pl.pallas_call(
        body,
        grid=(N // BN,),
        in_specs=[
            pl.BlockSpec((BN, H), lambda i: (i, 0)),
            pl.BlockSpec((BN, H), lambda i: (i, 0)),
            pl.BlockSpec((BN, XG), lambda i: (i, 0)),
            pl.BlockSpec((NC, BN, XG), lambda i: (0, i, 0)),
            _full_spec((H, H)),
            _full_spec((H, H)),
            _full_spec((1, H)),
            _full_spec((H, H)),
            _full_spec((1, H)),
        ] + extra_in,
        out_specs=out_specs,
        out_shape=out_shape,
        compiler_params=pltpu.CompilerParams(dimension_semantics=("parallel",)),
    )(hres, agg, xp, xaggp, Wn1a, Wn1b, bn1, Wn2, bn2, *extra_args)


@functools.lru_cache(maxsize=None)
def _make_gather(N, E, H):
    nch = E // CH
    assert nch % 2 == 0
    base_sz = (nch // NS) & ~1
    extra = nch - base_sz * NS
    E4 = E * 4 // XG
    mesh = plsc.VectorSubcoreMesh(core_axis_name="c", subcore_axis_name="s",
                                  num_cores=NC, num_subcores=NS)

    @functools.partial(
        pl.kernel,
        out_type=(
            jax.ShapeDtypeStruct((E, H), _f32),
            jax.ShapeDtypeStruct((E, H), _f32),
            jax.ShapeDtypeStruct((E4, XG), _f32),
        ),
        mesh=mesh,
        scratch_types=[
            pltpu.VMEM((2, CH), _i32),
            pltpu.VMEM((2, CH), _i32),
            pltpu.VMEM((2, CH, H), _f32),
            pltpu.VMEM((8, XG), _f32),
            pltpu.VMEM((N,), _f32),
            pltpu.VMEM((N,), _f32),
            pltpu.VMEM((N,), _f32),
            pltpu.SemaphoreType.DMA,
            pltpu.SemaphoreType.DMA,
            pltpu.SemaphoreType.DMA,
        ],
        compiler_params=pltpu.CompilerParams(needs_layout_passes=False),
    )
    def gather(hs, hd, xx, xy, xz, src, dst, hsg, hdg, xrp,
               idxs, idxd, gbuf, xrbuf, xxv, xyv, xzv, sa, sb, sx):
        c = lax.axis_index("c")
        s = lax.axis_index("s")
        sz = base_sz + jnp.where(s == 0, extra, 0)
        start = base_sz * s + jnp.where(s > 0, extra, 0)
        npairs = sz // 2
        lane = lax.iota(_i32, LN)

        @pl.when(c == 0)
        def _():
            pltpu.sync_copy(xx, xxv)
            pltpu.sync_copy(xy, xyv)
            pltpu.sync_copy(xz, xzv)

            def body(j, carry):
                t0 = start + 2 * j
                base0 = t0 * CH
                base1 = base0 + CH
                pltpu.sync_copy(src.at[pl.ds(base0, CH)], idxs.at[0])
                pltpu.sync_copy(dst.at[pl.ds(base0, CH)], idxd.at[0])
                pltpu.sync_copy(src.at[pl.ds(base1, CH)], idxs.at[1])
                pltpu.sync_copy(dst.at[pl.ds(base1, CH)], idxd.at[1])
                dA = pltpu.async_copy(hs.at[idxs.at[0]], gbuf.at[0], sa)
                dB = pltpu.async_copy(hs.at[idxs.at[1]], gbuf.at[1], sb)
                for half in range(2):
                    for g in range(CH // LN):
                        iv_s = idxs[half, pl.ds(g * LN, LN)]
                        iv_d = idxd[half, pl.ds(g * LN, LN)]
                        ax = plsc.load_gather(xxv, [iv_s]) - plsc.load_gather(xxv, [iv_d])
                        ay = plsc.load_gather(xyv, [iv_s]) - plsc.load_gather(xyv, [iv_d])
                        az = plsc.load_gather(xzv, [iv_s]) - plsc.load_gather(xzv, [iv_d])
                        d2v = ax * ax + ay * ay + az * az
                        row = jnp.full((LN,), half * 4 + g // 2, _i32)
                        colb = lane * 4 + (g % 2) * 64
                        plsc.store_scatter(xrbuf, [row, colb], ax)
                        plsc.store_scatter(xrbuf, [row, colb + 1], ay)
                        plsc.store_scatter(xrbuf, [row, colb + 2], az)
                        plsc.store_scatter(xrbuf, [row, colb + 3], d2v)
                dA.wait()
                eA = pltpu.async_copy(gbuf.at[0], hsg.at[pl.ds(base0, CH)], sa)
                dB.wait()
                eB = pltpu.async_copy(gbuf.at[1], hsg.at[pl.ds(base1, CH)], sb)
                eX = pltpu.async_copy(xrbuf, xrp.at[pl.ds(t0 * 4, 8)], sx)
                eA.wait()
                eB.wait()
                eX.wait()
                return carry

            lax.fori_loop(0, npairs, body, 0)

        @pl.when(c == 1)
        def _():
            def body(j, carry):
                t0 = start + 2 * j
                base0 = t0 * CH
                base1 = base0 + CH
                pltpu.sync_copy(dst.at[pl.ds(base0, CH)], idxd.at[0])
                pltpu.sync_copy(dst.at[pl.ds(base1, CH)], idxd.at[1])
                dA = pltpu.async_copy(hd.at[idxd.at[0]], gbuf.at[0], sa)
                dB = pltpu.async_copy(hd.at[idxd.at[1]], gbuf.at[1], sb)
                dA.wait()
                eA = pltpu.async_copy(gbuf.at[0], hdg.at[pl.ds(base0, CH)], sa)
                dB.wait()
                eB = pltpu.async_copy(gbuf.at[1], hdg.at[pl.ds(base1, CH)], sb)
                eA.wait()
                eB.wait()
                return carry

            lax.fori_loop(0, npairs, body, 0)

    return gather


@functools.lru_cache(maxsize=None)
def _make_scatter(N, E, H):
    HC = H // NC
    SP = -(-N // (NS * 8)) * 8
    N2 = SP * NS
    nch = E // CH
    base_a = (nch // NS) & ~1
    extra_a = nch - base_a * NS
    nchb = nch // NC
    base_b = (nchb // NS) & ~1
    extra_b = nchb - base_b * NS
    mesh = plsc.VectorSubcoreMesh(core_axis_name="c", subcore_axis_name="s",
                                  num_cores=NC, num_subcores=NS)

    @functools.partial(
        pl.kernel,
        out_type=(
            jax.ShapeDtypeStruct((N2, H), _f32),
            jax.ShapeDtypeStruct((NC, N2, XG), _f32),
        ),
        mesh=mesh,
        scratch_types=[
            pltpu.VMEM((2, CH), _i32),
            pltpu.VMEM((2, CH, HC), _f32),
            pltpu.VMEM_SHARED((N2, HC), _f32),
            pltpu.SemaphoreType.DMA,
            pltpu.SemaphoreType.DMA,
        ],
    )
    def scatter(m, rc, dst, zeros, agg, xaggp, dstv, rows, acc, sa, sb):
        c = lax.axis_index("c")
        s = lax.axis_index("s")
        rbase = s * SP

        pltpu.sync_copy(zeros.at[pl.ds(rbase, SP)], acc.at[pl.ds(rbase, SP)])
        plsc.subcore_barrier()

        sz_a = base_a + jnp.where(s == 0, extra_a, 0)
        start_a = base_a * s + jnp.where(s > 0, extra_a, 0)

        def step_a(j, carry):
            base0 = (start_a + 2 * j) * CH
            base1 = base0 + CH
            pltpu.sync_copy(dst.at[pl.ds(base0, CH)], dstv.at[0])
            dA = pltpu.async_copy(m.at[pl.ds(base0, CH), pl.ds(c * HC, HC)],
                                  rows.at[0], sa)
            pltpu.sync_copy(dst.at[pl.ds(base1, CH)], dstv.at[1])
            dB = pltpu.async_copy(m.at[pl.ds(base1, CH), pl.ds(c * HC, HC)],
                                  rows.at[1], sb)
            dA.wait()
            pltpu.sync_copy(rows.at[0], acc.at[dstv.at[0]], add=True)
            dB.wait()
            pltpu.sync_copy(rows.at[1], acc.at[dstv.at[1]], add=True)
            return carry

        lax.fori_loop(0, sz_a // 2, step_a, 0)
        plsc.subcore_barrier()

        pltpu.sync_copy(acc.at[pl.ds(rbase, SP)],
                        agg.at[pl.ds(rbase, SP), pl.ds(c * HC, HC)])
        pltpu.sync_copy(zeros.at[pl.ds(rbase, SP)], acc.at[pl.ds(rbase, SP)])
        plsc.subcore_barrier()

        sz_b = base_b + jnp.where(s == 0, extra_b, 0)
        start_b = c * nchb + base_b * s + jnp.where(s > 0, extra_b, 0)

        def step_b(j, carry):
            base0 = (start_b + 2 * j) * CH
            base1 = base0 + CH
            pltpu.sync_copy(dst.at[pl.ds(base0, CH)], dstv.at[0])
            dA = pltpu.async_copy(rc.at[pl.ds(base0, CH)], rows.at[0], sa)
            pltpu.sync_copy(dst.at[pl.ds(base1, CH)], dstv.at[1])
            dB = pltpu.async_copy(rc.at[pl.ds(base1, CH)], rows.at[1], sb)
            dA.wait()
            pltpu.sync_copy(rows.at[0], acc.at[dstv.at[0]], add=True)
            dB.wait()
            pltpu.sync_copy(rows.at[1], acc.at[dstv.at[1]], add=True)
            return carry

        lax.fori_loop(0, sz_b // 2, step_b, 0)

        @pl.when(sz_b % 2 == 1)
        def _():
            base0 = (start_b + (sz_b // 2) * 2) * CH
            pltpu.sync_copy(dst.at[pl.ds(base0, CH)], dstv.at[0])
            pltpu.sync_copy(rc.at[pl.ds(base0, CH)], rows.at[0])
            pltpu.sync_copy(rows.at[0], acc.at[dstv.at[0]], add=True)

        plsc.subcore_barrier()

        pltpu.sync_copy(acc.at[pl.ds(rbase, SP)],
                        xaggp.at[c, pl.ds(rbase, SP)])

    return scatter


def kernel(h, x, edge_index, W_in, b_in, W_out, b_out, We1, be1, We2, be2,
           Wn1, bn1, Wn2, bn2, Wx, bx):
    N, IN = h.shape
    E = edge_index.shape[1]
    H = W_in.shape[1]
    OUT = W_out.shape[1]
    DEPTH = We1.shape[0]

    src = edge_index[0]
    dst = edge_index[1]
    xp = jnp.zeros((N, XG), _f32).at[:, :3].set(x)

    A = We1[:, :H, :]
    B = We1[:, H:2 * H, :]
    wc = We1[:, 2 * H, :]
    SP = -(-N // (NS * 8)) * 8
    zeros = jnp.zeros((SP * NS, H // NC), _f32)

    gather = _make_gather(N, E, H)
    scatter = _make_scatter(N, E, H)

    hcur, hs, hd = _embed(h, W_in, b_in.reshape(1, H), A[0], B[0],
                          be1[0].reshape(1, H))

    hout = None
    for l in range(DEPTH):
        xx, xy, xz = xp[:, 0], xp[:, 1], xp[:, 2]
        hsg, hdg, xrp = gather(hs, hd, xx, xy, xz, src, dst)
        xr4 = xrp.reshape(E, 4)
        m2, rc = _edge_mlp(hsg, hdg, xr4, We2[l], be2[l].reshape(1, H),
                           wc[l].reshape(1, H), Wx[l].reshape(1, H),
                           bx[l].reshape(1, 1))
        agg, xaggp = scatter(m2, rc, dst, zeros)
        Wn1a = Wn1[l, :H]
        Wn1b = Wn1[l, H:]
        if l < DEPTH - 1:
            hcur, xp, hs, hd = _node_update(
                hcur, agg, xp, xaggp, Wn1a, Wn1b, bn1[l].reshape(1, H), Wn2[l],
                bn2[l].reshape(1, H),
                (A[l + 1], B[l + 1], be1[l + 1].reshape(1, H)), last=False)
        else:
            hout, xp = _node_update(
                hcur, agg, xp, xaggp, Wn1a, Wn1b, bn1[l].reshape(1, H), Wn2[l],
                bn2[l].reshape(1, H), (W_out, b_out.reshape(1, OUT)), last=True)

    return hout, xp[:, :3]


# speedup vs baseline: 1.1237x; 1.1237x over previous
import functools

import jax
import jax.numpy as jnp
from jax import lax
from jax.experimental import pallas as pl
from jax.experimental.pallas import tpu as pltpu
from jax.experimental.pallas import tpu_sc as plsc

NC = 2
NS = 16
NW = NC * NS
LN = 16
CH = 128
XW = 16
XG = 128

_f32 = jnp.float32
_i32 = jnp.int32


def _silu(v):
    return v * jax.nn.sigmoid(v)


def _dbf(a, b):
    return jnp.dot(a.astype(jnp.bfloat16), b.astype(jnp.bfloat16),
                   preferred_element_type=_f32)


def _tr(v):
    return v.astype(jnp.bfloat16).astype(_f32)


def _full_spec(shape):
    nd = len(shape)
    return pl.BlockSpec(shape, lambda i, _nd=nd: (0,) * _nd)


def _embed(h, W_in, b_in, A0, B0, be10, BN=1000):
    N, IN = h.shape
    H = W_in.shape[1]

    def body(h_ref, wi_ref, bi_ref, a_ref, b_ref, be_ref, he_ref, hs_ref, hd_ref):
        he = _dbf(h_ref[...], wi_ref[...]) + bi_ref[...]
        he_ref[...] = he
        hs_ref[...] = _dbf(he, a_ref[...]) + be_ref[...]
        hd_ref[...] = _dbf(he, b_ref[...])

    return pl.pallas_call(
        body,
        grid=(N // BN,),
        in_specs=[
            pl.BlockSpec((BN, IN), lambda i: (i, 0)),
            _full_spec((IN, H)),
            _full_spec((1, H)),
            _full_spec((H, H)),
            _full_spec((H, H)),
            _full_spec((1, H)),
        ],
        out_specs=[
            pl.BlockSpec((BN, H), lambda i: (i, 0)),
            pl.BlockSpec((BN, H), lambda i: (i, 0)),
            pl.BlockSpec((BN, H), lambda i: (i, 0)),
        ],
        out_shape=[jax.ShapeDtypeStruct((N, H), _f32)] * 3,
        compiler_params=pltpu.CompilerParams(dimension_semantics=("parallel",)),
    )(h, W_in, b_in, A0, B0, be10)


def _edge_mlp(hsg, hdg, xr, We2, be2, wc, wx, bx, BE=1000):
    E, H = hsg.shape

    def body(hs_ref, hd_ref, xr_ref, w2_ref, b2_ref, wc_ref, wx_ref, bx_ref,
             m_ref, rc_ref):
        xrv = xr_ref[...]
        d2 = xrv[:, 3:4]
        col = lax.broadcasted_iota(_i32, (1, XG), 1)
        r = xrv * (col < 3).astype(_f32)
        pre = hs_ref[...] + hd_ref[...] + _tr(d2) * _tr(wc_ref[...])
        m1 = _silu(pre)
        m2 = _silu(_dbf(m1, w2_ref[...]) + b2_ref[...])
        m_ref[...] = m2
        t = jnp.sum(_tr(m2) * _tr(wx_ref[...]), axis=1, keepdims=True) + bx_ref[...]
        coef = jnp.tanh(t)
        rc_ref[...] = r * coef + (col == 3).astype(_f32)

    return pl.pallas_call(
        body,
        grid=(E // BE,),
        in_specs=[
            pl.BlockSpec((BE, H), lambda i: (i, 0)),
            pl.BlockSpec((BE, H), lambda i: (i, 0)),
            pl.BlockSpec((BE, XG), lambda i: (i, 0)),
            _full_spec((H, H)),
            _full_spec((1, H)),
            _full_spec((1, H)),
            _full_spec((1, H)),
            _full_spec((1, 1)),
        ],
        out_specs=[
            pl.BlockSpec((BE, H), lambda i: (i, 0)),
            pl.BlockSpec((BE, XG), lambda i: (i, 0)),
        ],
        out_shape=[
            jax.ShapeDtypeStruct((E, H), _f32),
            jax.ShapeDtypeStruct((E, XG), _f32),
        ],
        compiler_params=pltpu.CompilerParams(dimension_semantics=("parallel",)),
    )(hsg, hdg, xr, We2, be2, wc, wx, bx)


def _node_update(hres, agg, xp, xaggp, Wn1a, Wn1b, bn1, Wn2, bn2, Wnext, last,
                 BN=1000):
    N, H = hres.shape

    def _common(h_ref, agg_ref, x_ref, xa_ref, wa_ref, wb_ref, b1_ref, w2_ref,
                b2_ref):
        hres_v = h_ref[...]
        hn1 = _silu(_dbf(hres_v, wa_ref[...])
                    + _dbf(agg_ref[...], wb_ref[...])
                    + b1_ref[...])
        hn2 = _dbf(hn1, w2_ref[...]) + b2_ref[...]
        hnew = _silu(hn2) + hres_v
        xa = xa_ref[0] + xa_ref[1]
        inv = 1.0 / (xa[:, 3:4] + 1.0)
        col = lax.broadcasted_iota(_i32, (1, XG), 1)
        mask3 = (col < 3).astype(_f32)
        xnew = x_ref[...] + xa * inv * mask3
        return hnew, xnew

    if last:
        W_out, b_out = Wnext
        OUT = W_out.shape[1]

        def body(h_ref, agg_ref, x_ref, xa_ref, wa_ref, wb_ref, b1_ref, w2_ref,
                 b2_ref, wo_ref, bo_ref, ho_ref, xo_ref):
            hnew, xnew = _common(h_ref, agg_ref, x_ref, xa_ref, wa_ref, wb_ref,
                                 b1_ref, w2_ref, b2_ref)
            xo_ref[...] = xnew
            ho_ref[...] = _dbf(hnew, wo_ref[...]) + bo_ref[...]

        extra_in = [_full_spec((H, OUT)), _full_spec((1, OUT))]
        extra_args = [W_out, b_out]
        out_specs = [
            pl.BlockSpec((BN, OUT), lambda i: (i, 0)),
            pl.BlockSpec((BN, XG), lambda i: (i, 0)),
        ]
        out_shape = [
            jax.ShapeDtypeStruct((N, OUT), _f32),
            jax.ShapeDtypeStruct((N, XG), _f32),
        ]
    else:
        An, Bn, ben = Wnext

        def body(h_ref, agg_ref, x_ref, xa_ref, wa_ref, wb_ref, b1_ref, w2_ref,
                 b2_ref, an_ref, bnn_ref, ben_ref, hn_ref, xo_ref, hs_ref, hd_ref):
            hnew, xnew = _common(h_ref, agg_ref, x_ref, xa_ref, wa_ref, wb_ref,
                                 b1_ref, w2_ref, b2_ref)
            hn_ref[...] = hnew
            xo_ref[...] = xnew
            hs_ref[...] = _dbf(hnew, an_ref[...]) + ben_ref[...]
            hd_ref[...] = _dbf(hnew, bnn_ref[...])

        extra_in = [_full_spec((H, H)), _full_spec((H, H)), _full_spec((1, H))]
        extra_args = [An, Bn, ben]
        out_specs = [
            pl.BlockSpec((BN, H), lambda i: (i, 0)),
            pl.BlockSpec((BN, XG), lambda i: (i, 0)),
            pl.BlockSpec((BN, H), lambda i: (i, 0)),
            pl.BlockSpec((BN, H), lambda i: (i, 0)),
        ]
        out_shape = [
            jax.ShapeDtypeStruct((N, H), _f32),
            jax.ShapeDtypeStruct((N, XG), _f32),
            jax.ShapeDtypeStruct((N, H), _f32),
            jax.ShapeDtypeStruct((N, H), _f32),
        ]

    return pl.pallas_call(
        body,
        grid=(N // BN,),
        in_specs=[
            pl.BlockSpec((BN, H), lambda i: (i, 0)),
            pl.BlockSpec((BN, H), lambda i: (i, 0)),
            pl.BlockSpec((BN, XG), lambda i: (i, 0)),
            pl.BlockSpec((NC, BN, XG), lambda i: (0, i, 0)),
            _full_spec((H, H)),
            _full_spec((H, H)),
            _full_spec((1, H)),
            _full_spec((H, H)),
            _full_spec((1, H)),
        ] + extra_in,
        out_specs=out_specs,
        out_shape=out_shape,
        compiler_params=pltpu.CompilerParams(dimension_semantics=("parallel",)),
    )(hres, agg, xp, xaggp, Wn1a, Wn1b, bn1, Wn2, bn2, *extra_args)


@functools.lru_cache(maxsize=None)
def _make_gather(N, E, H):
    nch = E // CH
    rem = nch % NW
    ntb = nch // NW
    mesh = plsc.VectorSubcoreMesh(core_axis_name="c", subcore_axis_name="s",
                                  num_cores=NC, num_subcores=NS)

    @functools.partial(
        pl.kernel,
        out_type=(
            jax.ShapeDtypeStruct((E, H), _f32),
            jax.ShapeDtypeStruct((E, H), _f32),
            jax.ShapeDtypeStruct((E, XG), _f32),
        ),
        mesh=mesh,
        scratch_types=[
            pltpu.VMEM((CH,), _i32),
            pltpu.VMEM((CH,), _i32),
            pltpu.VMEM((CH, H), _f32),
            pltpu.VMEM((CH, H), _f32),
            pltpu.VMEM((CH, XG), _f32),
            pltpu.VMEM((N,), _f32),
            pltpu.VMEM((N,), _f32),
            pltpu.VMEM((N,), _f32),
            pltpu.SemaphoreType.DMA,
            pltpu.SemaphoreType.DMA,
            pltpu.SemaphoreType.DMA,
        ],
        compiler_params=pltpu.CompilerParams(needs_layout_passes=False),
    )
    def gather(hs, hd, xx, xy, xz, src, dst, hsg, hdg, xr,
               srcv, dstv, ra, rb, xrbuf, xxv, xyv, xzv, s1, s2, s3):
        wid = lax.axis_index("s") * NC + lax.axis_index("c")
        nt = ntb + jnp.where(wid < rem, 1, 0)

        pltpu.sync_copy(xx, xxv)
        pltpu.sync_copy(xy, xyv)
        pltpu.sync_copy(xz, xzv)

        zlane = jnp.zeros((LN,), _f32)

        def zinit(i, carry):
            for j in range(XG // LN):
                xrbuf[i, pl.ds(j * LN, LN)] = zlane
            return carry

        lax.fori_loop(0, CH, zinit, 0)

        def step(i, carry):
            t = wid + i * NW
            base = t * CH
            pltpu.sync_copy(src.at[pl.ds(base, CH)], srcv)
            pltpu.sync_copy(dst.at[pl.ds(base, CH)], dstv)
            d1 = pltpu.async_copy(hs.at[srcv], ra, s1)
            d2 = pltpu.async_copy(hd.at[dstv], rb, s2)
            lane = lax.iota(_i32, LN)
            for g in range(CH // LN):
                idxs = srcv[pl.ds(g * LN, LN)]
                idxd = dstv[pl.ds(g * LN, LN)]
                ax = plsc.load_gather(xxv, [idxs]) - plsc.load_gather(xxv, [idxd])
                ay = plsc.load_gather(xyv, [idxs]) - plsc.load_gather(xyv, [idxd])
                az = plsc.load_gather(xzv, [idxs]) - plsc.load_gather(xzv, [idxd])
                d2v = ax * ax + ay * ay + az * az
                rows = g * LN + lane
                plsc.store_scatter(xrbuf, [rows, jnp.full((LN,), 0, _i32)], ax)
                plsc.store_scatter(xrbuf, [rows, jnp.full((LN,), 1, _i32)], ay)
                plsc.store_scatter(xrbuf, [rows, jnp.full((LN,), 2, _i32)], az)
                plsc.store_scatter(xrbuf, [rows, jnp.full((LN,), 3, _i32)], d2v)
            d1.wait()
            d2.wait()
            e1 = pltpu.async_copy(ra, hsg.at[pl.ds(base, CH)], s1)
            e2 = pltpu.async_copy(rb, hdg.at[pl.ds(base, CH)], s2)
            e3 = pltpu.async_copy(xrbuf, xr.at[pl.ds(base, CH)], s3)
            e1.wait()
            e2.wait()
            e3.wait()
            return carry

        lax.fori_loop(0, nt, step, 0)

    return gather


@functools.lru_cache(maxsize=None)
def _make_scatter(N, E, H):
    HC = H // NC
    SP = -(-N // (NS * 8)) * 8
    N2 = SP * NS
    nch = E // CH
    base_a = (nch // NS) & ~1
    extra_a = nch - base_a * NS
    nchb = nch // NC
    base_b = (nchb // NS) & ~1
    extra_b = nchb - base_b * NS
    mesh = plsc.VectorSubcoreMesh(core_axis_name="c", subcore_axis_name="s",
                                  num_cores=NC, num_subcores=NS)

    @functools.partial(
        pl.kernel,
        out_type=(
            jax.ShapeDtypeStruct((N2, H), _f32),
            jax.ShapeDtypeStruct((NC, N2, XG), _f32),
        ),
        mesh=mesh,
        scratch_types=[
            pltpu.VMEM((2, CH), _i32),
            pltpu.VMEM((2, CH, HC), _f32),
            pltpu.VMEM_SHARED((N2, HC), _f32),
            pltpu.SemaphoreType.DMA,
            pltpu.SemaphoreType.DMA,
        ],
    )
    def scatter(m, rc, dst, zeros, agg, xaggp, dstv, rows, acc, sa, sb):
        c = lax.axis_index("c")
        s = lax.axis_index("s")
        rbase = s * SP

        pltpu.sync_copy(zeros.at[pl.ds(rbase, SP)], acc.at[pl.ds(rbase, SP)])
        plsc.subcore_barrier()

        sz_a = base_a + jnp.where(s == 0, extra_a, 0)
        start_a = base_a * s + jnp.where(s > 0, extra_a, 0)

        def step_a(j, carry):
            base0 = (start_a + 2 * j) * CH
            base1 = base0 + CH
            pltpu.sync_copy(dst.at[pl.ds(base0, CH)], dstv.at[0])
            dA = pltpu.async_copy(m.at[pl.ds(base0, CH), pl.ds(c * HC, HC)],
                                  rows.at[0], sa)
            pltpu.sync_copy(dst.at[pl.ds(base1, CH)], dstv.at[1])
            dB = pltpu.async_copy(m.at[pl.ds(base1, CH), pl.ds(c * HC, HC)],
                                  rows.at[1], sb)
            dA.wait()
            pltpu.sync_copy(rows.at[0], acc.at[dstv.at[0]], add=True)
            dB.wait()
            pltpu.sync_copy(rows.at[1], acc.at[dstv.at[1]], add=True)
            return carry

        lax.fori_loop(0, sz_a // 2, step_a, 0)
        plsc.subcore_barrier()

        pltpu.sync_copy(acc.at[pl.ds(rbase, SP)],
                        agg.at[pl.ds(rbase, SP), pl.ds(c * HC, HC)])
        pltpu.sync_copy(zeros.at[pl.ds(rbase, SP)], acc.at[pl.ds(rbase, SP)])
        plsc.subcore_barrier()

        sz_b = base_b + jnp.where(s == 0, extra_b, 0)
        start_b = c * nchb + base_b * s + jnp.where(s > 0, extra_b, 0)

        def step_b(j, carry):
            base0 = (start_b + 2 * j) * CH
            base1 = base0 + CH
            pltpu.sync_copy(dst.at[pl.ds(base0, CH)], dstv.at[0])
            dA = pltpu.async_copy(rc.at[pl.ds(base0, CH)], rows.at[0], sa)
            pltpu.sync_copy(dst.at[pl.ds(base1, CH)], dstv.at[1])
            dB = pltpu.async_copy(rc.at[pl.ds(base1, CH)], rows.at[1], sb)
            dA.wait()
            pltpu.sync_copy(rows.at[0], acc.at[dstv.at[0]], add=True)
            dB.wait()
            pltpu.sync_copy(rows.at[1], acc.at[dstv.at[1]], add=True)
            return carry

        lax.fori_loop(0, sz_b // 2, step_b, 0)

        @pl.when(sz_b % 2 == 1)
        def _():
            base0 = (start_b + (sz_b // 2) * 2) * CH
            pltpu.sync_copy(dst.at[pl.ds(base0, CH)], dstv.at[0])
            pltpu.sync_copy(rc.at[pl.ds(base0, CH)], rows.at[0])
            pltpu.sync_copy(rows.at[0], acc.at[dstv.at[0]], add=True)

        plsc.subcore_barrier()

        pltpu.sync_copy(acc.at[pl.ds(rbase, SP)],
                        xaggp.at[c, pl.ds(rbase, SP)])

    return scatter


def kernel(h, x, edge_index, W_in, b_in, W_out, b_out, We1, be1, We2, be2,
           Wn1, bn1, Wn2, bn2, Wx, bx):
    N, IN = h.shape
    E = edge_index.shape[1]
    H = W_in.shape[1]
    OUT = W_out.shape[1]
    DEPTH = We1.shape[0]

    src = edge_index[0]
    dst = edge_index[1]
    xp = jnp.zeros((N, XG), _f32).at[:, :3].set(x)

    A = We1[:, :H, :]
    B = We1[:, H:2 * H, :]
    wc = We1[:, 2 * H, :]
    SP = -(-N // (NS * 8)) * 8
    zeros = jnp.zeros((SP * NS, H // NC), _f32)

    gather = _make_gather(N, E, H)
    scatter = _make_scatter(N, E, H)

    hcur, hs, hd = _embed(h, W_in, b_in.reshape(1, H), A[0], B[0],
                          be1[0].reshape(1, H))

    hout = None
    for l in range(DEPTH):
        xx, xy, xz = xp[:, 0], xp[:, 1], xp[:, 2]
        hsg, hdg, xr = gather(hs, hd, xx, xy, xz, src, dst)
        m2, rc = _edge_mlp(hsg, hdg, xr, We2[l], be2[l].reshape(1, H),
                           wc[l].reshape(1, H), Wx[l].reshape(1, H),
                           bx[l].reshape(1, 1))
        agg, xaggp = scatter(m2, rc, dst, zeros)
        Wn1a = Wn1[l, :H]
        Wn1b = Wn1[l, H:]
        if l < DEPTH - 1:
            hcur, xp, hs, hd = _node_update(
                hcur, agg, xp, xaggp, Wn1a, Wn1b, bn1[l].reshape(1, H), Wn2[l],
                bn2[l].reshape(1, H),
                (A[l + 1], B[l + 1], be1[l + 1].reshape(1, H)), last=False)
        else:
            hout, xp = _node_update(
                hcur, agg, xp, xaggp, Wn1a, Wn1b, bn1[l].reshape(1, H), Wn2[l],
                bn2[l].reshape(1, H), (W_out, b_out.reshape(1, OUT)), last=True)

    return hout, xp[:, :3]
